# baseline (speedup 1.0000x reference)
import jax
import jax.numpy as jnp
from jax.experimental import pallas as pl
from jax.experimental.pallas import tpu as pltpu

M = 1_000_000
D = 64
R = 8192


def _tc_body(u_ref, x_ref, o_ref):
    x = x_ref[...]
    u = u_ref[...]
    o_ref[...] = jnp.sum(x * u, axis=1)


@jax.jit
def _tc_matvec(items_emb, user_emb):
    grid = ((M + R - 1) // R,)
    return pl.pallas_call(
        _tc_body,
        grid=grid,
        in_specs=[
            pl.BlockSpec((1, D), lambda i: (0, 0)),
            pl.BlockSpec((R, D), lambda i: (i, 0)),
        ],
        out_specs=pl.BlockSpec((R,), lambda i: (i,)),
        out_shape=jax.ShapeDtypeStruct((M,), jnp.float32),
    )(user_emb, items_emb)


def kernel(items_emb, user_emb):
    return _tc_matvec(items_emb, user_emb)
